# baseline (speedup 1.0000x reference)
import functools

import jax
import jax.numpy as jnp
from jax import lax
from jax.experimental import pallas as pl
from jax.experimental.pallas import tpu as pltpu
from jax.experimental.pallas import tpu_sc as plsc

_NC = 2
_NS = 16
_NW = _NC * _NS

_NBUF = 6
_RPG = 4
_LOOKAHEAD = 3


def _make_gather(b: int, l: int, embed: int):
    mesh = plsc.VectorSubcoreMesh(
        core_axis_name="c", subcore_axis_name="s",
        num_cores=_NC, num_subcores=_NS,
    )
    rows_per_w = b // _NW
    n_iters = rows_per_w // _RPG
    slot_rows = _RPG * l

    @functools.partial(
        pl.kernel,
        mesh=mesh,
        out_type=jax.ShapeDtypeStruct((b, l, embed), jnp.float32),
        scratch_types=[
            pltpu.VMEM((rows_per_w, l), jnp.int32),
            pltpu.VMEM((_NBUF, _RPG, l, embed), jnp.float32),
            pltpu.SemaphoreType.DMA,
            pltpu.SemaphoreType.DMA,
        ],
        compiler_params=pltpu.CompilerParams(use_tc_tiling_on_sc=False),
    )
    def gather_kernel(tok_hbm, w_hbm, out_hbm, idx_v, rows_v, gsem, ssem):
        wid = lax.axis_index("s") * _NC + lax.axis_index("c")
        tbase = wid * rows_per_w
        pltpu.sync_copy(tok_hbm.at[pl.ds(tbase, rows_per_w)], idx_v)

        def gather_descs(k):
            slot = lax.rem(k, _NBUF)
            return [
                pltpu.make_async_copy(
                    w_hbm.at[idx_v.at[k * _RPG + r]],
                    rows_v.at[slot, r],
                    gsem,
                )
                for r in range(_RPG)
            ]

        def scatter_desc(k):
            slot = lax.rem(k, _NBUF)
            return pltpu.make_async_copy(
                rows_v.at[slot],
                out_hbm.at[pl.ds(tbase + k * _RPG, _RPG)],
                ssem,
            )

        for k in range(_LOOKAHEAD):
            for d in gather_descs(k):
                d.start()

        def step(j, _):
            for d in gather_descs(j):
                d.wait()
            scatter_desc(j).start()

            @pl.when(j >= _LOOKAHEAD)
            def _():
                scatter_desc(j - _LOOKAHEAD).wait()

            @pl.when(j + _LOOKAHEAD < n_iters)
            def _():
                for d in gather_descs(j + _LOOKAHEAD):
                    d.start()

            return 0

        lax.fori_loop(0, n_iters, step, 0, unroll=False)

        for k in range(n_iters - _LOOKAHEAD, n_iters):
            scatter_desc(k).wait()

    return gather_kernel


def kernel(tokens, weight):
    b, l = tokens.shape
    vocab, embed = weight.shape
    assert b % (_NW * _RPG) == 0

    tok = tokens.astype(jnp.int32)
    return _make_gather(b, l, embed)(tok, weight)


# speedup vs baseline: 1.0533x; 1.0533x over previous
import functools

import jax
import jax.numpy as jnp
from jax import lax
from jax.experimental import pallas as pl
from jax.experimental.pallas import tpu as pltpu
from jax.experimental.pallas import tpu_sc as plsc

_NC = 2
_NS = 16
_NW = _NC * _NS

_NBUF = 6
_RPG = 4
_LOOKAHEAD = 3


def _make_gather(b: int, l: int, embed: int):
    mesh = plsc.VectorSubcoreMesh(
        core_axis_name="c", subcore_axis_name="s",
        num_cores=_NC, num_subcores=_NS,
    )
    rows_per_w = b // _NW
    n_iters = rows_per_w // _RPG
    slot_rows = _RPG * l

    @functools.partial(
        pl.kernel,
        mesh=mesh,
        out_type=jax.ShapeDtypeStruct((b, l, embed), jnp.float32),
        scratch_types=[
            pltpu.VMEM((rows_per_w, l), jnp.int32),
            pltpu.VMEM((_NBUF, _RPG, l, embed), jnp.float32),
            pltpu.SemaphoreType.DMA,
            pltpu.SemaphoreType.DMA,
        ],
        compiler_params=pltpu.CompilerParams(use_tc_tiling_on_sc=False),
    )
    def gather_kernel(tok_hbm, w_hbm, out_hbm, idx_v, rows_v, gsem, ssem):
        wid = lax.axis_index("s") * _NC + lax.axis_index("c")
        tbase = wid * rows_per_w
        pltpu.sync_copy(tok_hbm.at[pl.ds(tbase, rows_per_w)], idx_v)

        def gather_descs(k):
            slot = lax.rem(k, _NBUF)
            return [
                pltpu.make_async_copy(
                    w_hbm.at[idx_v.at[k * _RPG + r]],
                    rows_v.at[slot, r],
                    gsem,
                )
                for r in range(_RPG)
            ]

        def scatter_desc(k):
            slot = lax.rem(k, _NBUF)
            return pltpu.make_async_copy(
                rows_v.at[slot],
                out_hbm.at[pl.ds(tbase + k * _RPG, _RPG)],
                ssem,
            )

        for k in range(_LOOKAHEAD):
            for d in gather_descs(k):
                d.start()

        def step(j, _):
            for d in gather_descs(j):
                d.wait()
            scatter_desc(j).start()

            @pl.when(j >= _LOOKAHEAD)
            def _():
                scatter_desc(j - _LOOKAHEAD).wait()

            @pl.when(j + _LOOKAHEAD < n_iters)
            def _():
                for d in gather_descs(j + _LOOKAHEAD):
                    d.start()

            return 0

        lax.fori_loop(0, n_iters, step, 0, unroll=False)

        for k in range(n_iters - _LOOKAHEAD, n_iters):
            scatter_desc(k).wait()

    return gather_kernel


def kernel(tokens, weight):
    b, l = tokens.shape
    vocab, embed = weight.shape
    assert b % (_NW * _RPG) == 0

    tok = tokens.astype(jnp.int32) * 2
    w2 = jnp.pad(weight, ((0, 0), (0, 128 - embed))).reshape(2 * vocab, embed)
    return _make_gather(b, l, embed)(tok, w2)
